# Initial kernel scaffold; baseline (speedup 1.0000x reference)
import jax
import jax.numpy as jnp
from jax.experimental import pallas as pl


def kernel(x, W, b, keys, values, epsilons):
    raise NotImplementedError("write your pallas kernel here")


# speedup vs baseline: 1.1612x; 1.1612x over previous
import functools

import jax
import jax.numpy as jnp
from jax.experimental import pallas as pl
from jax.experimental.pallas import tpu as pltpu

B, S, D_IN, D_OUT, K = 4, 2048, 1024, 1024, 8192
TK = 1024
TS = 256
NKT = K // TK
NST = S // TS


def _retrieval_kernel(keys_ref, q_ref, eps_ref, vals_ref,
                      idx_out, mask_out, val_out, dmin_s, imin_s):
    t = pl.program_id(0)

    @pl.when(t == 0)
    def _init():
        dmin_s[...] = jnp.full(dmin_s.shape, jnp.inf, jnp.float32)
        imin_s[...] = jnp.zeros(imin_s.shape, jnp.int32)

    kb = keys_ref[...]
    q = q_ref[...]
    rk = jnp.sum(kb * kb, axis=1, keepdims=True)
    qn = jnp.sum(q * q, axis=1)[None, :]
    cross = jax.lax.dot_general(
        kb, q, (((1,), (1,)), ((), ())),
        preferred_element_type=jnp.float32)
    d2 = rk + qn - 2.0 * cross

    lmin = jnp.min(d2, axis=0)
    rows = jax.lax.broadcasted_iota(jnp.int32, d2.shape, 0) + t * TK
    lidx = jnp.min(jnp.where(d2 == lmin[None, :], rows, K), axis=0)

    better = lmin < dmin_s[0]
    dmin_s[0] = jnp.where(better, lmin, dmin_s[0])
    imin_s[0] = jnp.where(better, lidx, imin_s[0])

    @pl.when(t == NKT - 1)
    def _finish():
        idx = imin_s[0]
        dmin = dmin_s[0]
        kio = jax.lax.broadcasted_iota(jnp.int32, (K, B), 0)
        sel = kio == idx[None, :]
        eps = jnp.sum(jnp.where(sel, eps_ref[...], 0.0), axis=0)
        dist = jnp.sqrt(jnp.maximum(dmin, 0.0))
        idx_out[0] = idx
        mask_out[0] = (dist <= eps).astype(jnp.int32)
        val_out[...] = jax.lax.dot_general(
            sel.astype(jnp.float32), vals_ref[...],
            (((0,), (0,)), ((), ())),
            preferred_element_type=jnp.float32)


def _main_kernel(idx_ref, mask_ref, x_ref, w_ref, b_ref, v_ref, o_ref):
    bi = pl.program_id(0)
    m = mask_ref[bi]

    @pl.when(m == 1)
    def _replace():
        o_ref[0] = jnp.broadcast_to(v_ref[0], (TS, D_OUT))

    @pl.when(m == 0)
    def _matmul():
        acc = jax.lax.dot_general(
            x_ref[0], w_ref[...], (((1,), (1,)), ((), ())),
            preferred_element_type=jnp.float32)
        o_ref[0] = acc + b_ref[...]


@jax.jit
def kernel(x, W, b, keys, values, epsilons):
    query = x[:, -1, :]

    idx2, mask2, chosen_val = pl.pallas_call(
        _retrieval_kernel,
        grid=(NKT,),
        in_specs=[
            pl.BlockSpec((TK, D_IN), lambda t: (t, 0)),
            pl.BlockSpec((B, D_IN), lambda t: (0, 0)),
            pl.BlockSpec((K, 1), lambda t: (0, 0)),
            pl.BlockSpec((K, D_OUT), lambda t: (0, 0)),
        ],
        out_specs=[
            pl.BlockSpec((1, B), lambda t: (0, 0)),
            pl.BlockSpec((1, B), lambda t: (0, 0)),
            pl.BlockSpec((B, D_OUT), lambda t: (0, 0)),
        ],
        out_shape=[
            jax.ShapeDtypeStruct((1, B), jnp.int32),
            jax.ShapeDtypeStruct((1, B), jnp.int32),
            jax.ShapeDtypeStruct((B, D_OUT), jnp.float32),
        ],
        scratch_shapes=[
            pltpu.VMEM((1, B), jnp.float32),
            pltpu.VMEM((1, B), jnp.int32),
        ],
    )(keys, query, epsilons, values)

    idx = idx2.reshape(B)
    mask = mask2.reshape(B)

    grid_spec = pltpu.PrefetchScalarGridSpec(
        num_scalar_prefetch=2,
        grid=(B, NST),
        in_specs=[
            pl.BlockSpec(
                (1, TS, D_IN),
                lambda bi, si, idx_ref, mask_ref: (
                    bi, jnp.where(mask_ref[bi] == 1, 0, si), 0)),
            pl.BlockSpec((D_OUT, D_IN), lambda bi, si, idx_ref, mask_ref: (0, 0)),
            pl.BlockSpec((1, D_OUT), lambda bi, si, idx_ref, mask_ref: (0, 0)),
            pl.BlockSpec(
                (1, 1, D_OUT),
                lambda bi, si, idx_ref, mask_ref: (bi, 0, 0)),
        ],
        out_specs=pl.BlockSpec(
            (1, TS, D_OUT), lambda bi, si, idx_ref, mask_ref: (bi, si, 0)),
    )

    out = pl.pallas_call(
        _main_kernel,
        grid_spec=grid_spec,
        out_shape=jax.ShapeDtypeStruct((B, S, D_OUT), jnp.float32),
    )(idx, mask, x, W, b.reshape(1, D_OUT), chosen_val.reshape(B, 1, D_OUT))
    return out
